# baseline (speedup 1.0000x reference)
import functools

import jax
import jax.numpy as jnp
from jax import lax
from jax.experimental import pallas as pl
from jax.experimental.pallas import tpu as pltpu
from jax.experimental.pallas import tpu_sc as plsc

NUM_CORES = 2
NUM_SUBCORES = 16
LANES = 16
NW = NUM_CORES * NUM_SUBCORES

BATCH = 16384
B_PER_W = BATCH // NW
CHUNK = 128
NCHUNK = B_PER_W // CHUNK
STEPS = B_PER_W // LANES

SCALE = 0.005756462732485115


def _elo_body(x_hbm, r_hbm, out_hbm, xv, idx_i, idx_j, ri, rj, outv, sem):
    wid = lax.axis_index("s") * NUM_CORES + lax.axis_index("c")
    base = wid * B_PER_W

    pltpu.sync_copy(x_hbm.at[pl.ds(base * 3, B_PER_W * 3)], xv)

    lane = lax.iota(jnp.int32, LANES)

    def deint(t, _):
        pos = t * (3 * LANES) + lane * 3
        iv = plsc.load_gather(xv, [pos])
        jv = plsc.load_gather(xv, [pos + 1])
        r = t // 8
        c = (t % 8) * LANES
        idx_i[r, pl.ds(c, LANES)] = iv
        idx_j[r, pl.ds(c, LANES)] = jv
        return 0

    lax.fori_loop(0, STEPS, deint, 0, unroll=4)

    copies = []
    for r in range(NCHUNK):
        copies.append(pltpu.make_async_copy(r_hbm.at[idx_i.at[r]], ri.at[r], sem))
        copies.append(pltpu.make_async_copy(r_hbm.at[idx_j.at[r]], rj.at[r], sem))
    for cp in copies:
        cp.start()
    for cp in copies:
        cp.wait()

    def compute(t, _):
        r = t // 8
        c = (t % 8) * LANES
        a = ri[r, pl.ds(c, LANES)]
        b = rj[r, pl.ds(c, LANES)]
        outv[pl.ds(t * LANES, LANES)] = (a - b) * SCALE
        return 0

    lax.fori_loop(0, STEPS, compute, 0, unroll=4)

    pltpu.sync_copy(outv, out_hbm.at[pl.ds(base, B_PER_W)])


@jax.jit
def _elo_call(x_flat, r_flat):
    mesh = plsc.VectorSubcoreMesh(
        core_axis_name="c", subcore_axis_name="s",
        num_cores=NUM_CORES, num_subcores=NUM_SUBCORES)
    f = pl.kernel(
        _elo_body,
        out_type=jax.ShapeDtypeStruct((BATCH,), jnp.float32),
        mesh=mesh,
        compiler_params=pltpu.CompilerParams(needs_layout_passes=False),
        scratch_types=[
            pltpu.VMEM((B_PER_W * 3,), jnp.int32),
            pltpu.VMEM((NCHUNK, CHUNK), jnp.int32),
            pltpu.VMEM((NCHUNK, CHUNK), jnp.int32),
            pltpu.VMEM((NCHUNK, CHUNK), jnp.float32),
            pltpu.VMEM((NCHUNK, CHUNK), jnp.float32),
            pltpu.VMEM((B_PER_W,), jnp.float32),
            pltpu.SemaphoreType.DMA,
        ],
    )
    return f(x_flat, r_flat)


def kernel(x, R, Theta):
    del Theta
    out = _elo_call(x.reshape(-1), R.reshape(-1))
    return out.reshape(BATCH, 1)


# speedup vs baseline: 1.1644x; 1.1644x over previous
import jax
import jax.numpy as jnp
from jax import lax
from jax.experimental import pallas as pl
from jax.experimental.pallas import tpu as pltpu
from jax.experimental.pallas import tpu_sc as plsc

NUM_CORES = 2
NUM_SUBCORES = 16
LANES = 16
NW = NUM_CORES * NUM_SUBCORES

BATCH = 16384
B_PER_W = BATCH // NW
CHUNK = 128
NCHUNK = B_PER_W // CHUNK
STEPS = B_PER_W // LANES

SCALE = 0.005756462732485115
SCALE16 = SCALE * 16.0


def _elo_body(i_hbm, j_hbm, r_hbm, out_hbm, idx_i, idx_j, ri, rj, outv, sem):
    wid = lax.axis_index("s") * NUM_CORES + lax.axis_index("c")
    base = wid * B_PER_W

    idx_copies = []
    for r in range(NCHUNK):
        idx_copies.append(pltpu.make_async_copy(
            i_hbm.at[pl.ds(base + r * CHUNK, CHUNK)], idx_i.at[r], sem))
        idx_copies.append(pltpu.make_async_copy(
            j_hbm.at[pl.ds(base + r * CHUNK, CHUNK)], idx_j.at[r], sem))
    for cp in idx_copies:
        cp.start()
    for cp in idx_copies:
        cp.wait()

    gathers = []
    for r in range(NCHUNK):
        gathers.append(pltpu.make_async_copy(r_hbm.at[idx_i.at[r]], ri.at[r], sem))
        gathers.append(pltpu.make_async_copy(r_hbm.at[idx_j.at[r]], rj.at[r], sem))
    for cp in gathers:
        cp.start()
    for cp in gathers:
        cp.wait()

    def compute(t, _):
        r = t // 8
        c = (t % 8) * LANES
        a = ri[r, pl.ds(c, LANES)]
        b = rj[r, pl.ds(c, LANES)]
        outv[pl.ds(t * LANES, LANES)] = (a - b) * SCALE16
        return 0

    lax.fori_loop(0, STEPS, compute, 0, unroll=4)

    pltpu.sync_copy(outv, out_hbm.at[pl.ds(base, B_PER_W)])


@jax.jit
def _elo_call(i_idx, j_idx, r_table):
    mesh = plsc.VectorSubcoreMesh(
        core_axis_name="c", subcore_axis_name="s",
        num_cores=NUM_CORES, num_subcores=NUM_SUBCORES)
    f = pl.kernel(
        _elo_body,
        out_type=jax.ShapeDtypeStruct((BATCH,), jnp.float32),
        mesh=mesh,
        compiler_params=pltpu.CompilerParams(needs_layout_passes=False),
        scratch_types=[
            pltpu.VMEM((NCHUNK, CHUNK), jnp.int32),
            pltpu.VMEM((NCHUNK, CHUNK), jnp.int32),
            pltpu.VMEM((NCHUNK, CHUNK), jnp.float32),
            pltpu.VMEM((NCHUNK, CHUNK), jnp.float32),
            pltpu.VMEM((B_PER_W,), jnp.float32),
            pltpu.SemaphoreType.DMA,
        ],
    )
    return f(i_idx, j_idx, r_table)


def kernel(x, R, Theta):
    del Theta
    r_flat = (R * jnp.float32(0.0625)).reshape(-1)
    out = _elo_call(x[:, 0], x[:, 1], r_flat)
    return out.reshape(BATCH, 1)


# speedup vs baseline: 1.1649x; 1.0005x over previous
import jax
import jax.numpy as jnp
from jax import lax
from jax.experimental import pallas as pl
from jax.experimental.pallas import tpu as pltpu
from jax.experimental.pallas import tpu_sc as plsc

NUM_CORES = 2
NUM_SUBCORES = 16
LANES = 16
NW = NUM_CORES * NUM_SUBCORES

BATCH = 16384
B_PER_W = BATCH // NW
CHUNK = 128
NCHUNK = B_PER_W // CHUNK
STEPS = B_PER_W // LANES

SCALE = 0.005756462732485115


def _elo_body(i_hbm, j_hbm, r_hbm, out_hbm, idx_i, idx_j, ri, rj, outv, sem):
    wid = lax.axis_index("s") * NUM_CORES + lax.axis_index("c")
    base = wid * B_PER_W

    ci = pltpu.make_async_copy(i_hbm.at[pl.ds(base, B_PER_W)], idx_i, sem)
    cj = pltpu.make_async_copy(j_hbm.at[pl.ds(base, B_PER_W)], idx_j, sem)
    ci.start()
    cj.start()
    ci.wait()
    cj.wait()

    gathers = []
    for r in range(NCHUNK):
        gathers.append(pltpu.make_async_copy(
            r_hbm.at[idx_i.at[pl.ds(r * CHUNK, CHUNK)]],
            ri.at[pl.ds(r * CHUNK, CHUNK)], sem))
        gathers.append(pltpu.make_async_copy(
            r_hbm.at[idx_j.at[pl.ds(r * CHUNK, CHUNK)]],
            rj.at[pl.ds(r * CHUNK, CHUNK)], sem))
    for cp in gathers:
        cp.start()
    for cp in gathers:
        cp.wait()

    def compute(t, _):
        sl = pl.ds(t * LANES, LANES)
        outv[sl] = (ri[sl] - rj[sl]) * SCALE
        return 0

    lax.fori_loop(0, STEPS, compute, 0, unroll=4)

    pltpu.sync_copy(outv, out_hbm.at[pl.ds(base, B_PER_W)])


@jax.jit
def _elo_call(i_idx, j_idx, r_flat):
    mesh = plsc.VectorSubcoreMesh(
        core_axis_name="c", subcore_axis_name="s",
        num_cores=NUM_CORES, num_subcores=NUM_SUBCORES)
    f = pl.kernel(
        _elo_body,
        out_type=jax.ShapeDtypeStruct((BATCH,), jnp.float32),
        mesh=mesh,
        compiler_params=pltpu.CompilerParams(needs_layout_passes=False),
        scratch_types=[
            pltpu.VMEM((B_PER_W,), jnp.int32),
            pltpu.VMEM((B_PER_W,), jnp.int32),
            pltpu.VMEM((B_PER_W,), jnp.float32),
            pltpu.VMEM((B_PER_W,), jnp.float32),
            pltpu.VMEM((B_PER_W,), jnp.float32),
            pltpu.SemaphoreType.DMA,
        ],
    )
    return f(i_idx, j_idx, r_flat)


def kernel(x, R, Theta):
    del Theta
    out = _elo_call(x[:, 0], x[:, 1], R.reshape(-1))
    return out.reshape(BATCH, 1)
